# Initial kernel scaffold; baseline (speedup 1.0000x reference)
import jax
import jax.numpy as jnp
from jax.experimental import pallas as pl


def kernel(q, k):
    raise NotImplementedError("write your pallas kernel here")


# speedup vs baseline: 2.0514x; 2.0514x over previous
import jax
import jax.numpy as jnp
from jax.experimental import pallas as pl
from jax.experimental.pallas import tpu as pltpu

_TEMP = 45.254834
_BR = 256


def _fused_attn_kernel(q_ref, k_hbm, out_ref, k_vmem, sem):
    r = pl.program_id(0)

    @pl.when(r == 0)
    def _load_k():
        cp = pltpu.make_async_copy(k_hbm, k_vmem, sem)
        cp.start()
        cp.wait()

    logits = jax.lax.dot_general(
        q_ref[:], k_vmem[:],
        (((1,), (1,)), ((), ())),
        preferred_element_type=jnp.float32,
    ) * (1.0 / _TEMP)
    m = jnp.max(logits, axis=-1, keepdims=True)
    e = jnp.exp(logits - m)
    out_ref[:] = e / jnp.sum(e, axis=-1, keepdims=True)


def kernel(q, k):
    n, d = q.shape
    nk = k.shape[0]
    return pl.pallas_call(
        _fused_attn_kernel,
        grid=(n // _BR,),
        in_specs=[
            pl.BlockSpec((_BR, d), lambda r: (r, 0)),
            pl.BlockSpec(memory_space=pl.ANY),
        ],
        out_specs=pl.BlockSpec((_BR, nk), lambda r: (r, 0)),
        out_shape=jax.ShapeDtypeStruct((n, nk), jnp.float32),
        scratch_shapes=[
            pltpu.VMEM((nk, d), jnp.float32),
            pltpu.SemaphoreType.DMA,
        ],
        compiler_params=pltpu.CompilerParams(
            dimension_semantics=("arbitrary",),
            vmem_limit_bytes=100 * 1024 * 1024,
        ),
    )(q, k)
